# baseline (speedup 1.0000x reference)
import functools

import jax
import jax.numpy as jnp
from jax import lax
from jax.experimental import pallas as pl
from jax.experimental.pallas import tpu as pltpu

EPS = 1e-8


def _interp_body(x_ref, y_ref, xi_ref, out_ref):
    B = xi_ref.shape[0]
    R8 = x_ref.shape[1]
    lane = lax.broadcasted_iota(jnp.int32, (R8, 128), 1) % 16
    m0 = lane == 0
    m14 = lane == 14
    m15 = lane == 15
    e_col = lax.broadcasted_iota(jnp.int32, (8, 128), 1) // 16
    e_row = lax.broadcasted_iota(jnp.int32, (8, 128), 0)
    expand = jnp.where(e_col == e_row, 1.0, 0.0).astype(jnp.float32)
    c_row = lax.broadcasted_iota(jnp.int32, (128, 8), 0) // 16
    c_col = lax.broadcasted_iota(jnp.int32, (128, 8), 1)
    compact = jnp.where(c_row == c_col, 1.0, 0.0).astype(jnp.float32)
    for c in range(3):
        xb = x_ref[c]
        yb = y_ref[c]
        xn = pltpu.roll(xb, 127, 1)
        yn = pltpu.roll(yb, 127, 1)
        dxe = (xn - xb) + EPS
        dyr = (yn - yb) / dxe
        y0r = pltpu.roll(yb, 15, 1)
        for b in range(B):
            xiq = jnp.dot(xi_ref[b, c], expand, preferred_element_type=jnp.float32)
            t = xiq - xb
            tmin = jnp.minimum(t, dxe)
            tcl = jnp.maximum(tmin, 0.0)
            u = jnp.where(m0, tmin, tcl)
            u = jnp.where(m14, jnp.maximum(t, 0.0), u)
            w = u * dyr
            w = jnp.where(m15, y0r, w)
            out_ref[b, c] = jnp.dot(w, compact, preferred_element_type=jnp.float32)


def _mix_body(resp_ref, v_ref, out_ref):
    B = resp_ref.shape[0]
    for b in range(B):
        r0 = resp_ref[b, 0]
        r1 = resp_ref[b, 1]
        r2 = resp_ref[b, 2]
        for d in range(3):
            o = r0 * v_ref[0, d] + r1 * v_ref[1, d] + r2 * v_ref[2, d]
            out_ref[b, d] = jnp.clip(o, 0.0, 1.0)


@functools.partial(jax.jit, static_argnames=())
def kernel(input_image, V, x_data, y_data):
    B = input_image.shape[0]
    _, H, W, n = x_data.shape
    HW = H * W
    R8 = min(256, HW // 8)
    grid = (HW // 8) // R8

    x_pk = x_data.reshape(3, HW // 8, 128)
    y_pk = y_data.reshape(3, HW // 8, 128)
    xi = input_image.reshape(B, 3, HW // 8, 8)

    resp = pl.pallas_call(
        _interp_body,
        grid=(grid,),
        in_specs=[
            pl.BlockSpec((3, R8, 128), lambda i: (0, i, 0)),
            pl.BlockSpec((3, R8, 128), lambda i: (0, i, 0)),
            pl.BlockSpec((B, 3, R8, 8), lambda i: (0, 0, i, 0)),
        ],
        out_specs=pl.BlockSpec((B, 3, R8, 8), lambda i: (0, 0, i, 0)),
        out_shape=jax.ShapeDtypeStruct((B, 3, HW // 8, 8), jnp.float32),
    )(x_pk, y_pk, xi).reshape(B, 3, HW)

    v_t = jnp.transpose(V.reshape(HW, 3, 3), (1, 2, 0))
    Rk = min(4096, HW)
    out = pl.pallas_call(
        _mix_body,
        grid=(HW // Rk,),
        in_specs=[
            pl.BlockSpec((B, 3, Rk), lambda i: (0, 0, i)),
            pl.BlockSpec((3, 3, Rk), lambda i: (0, 0, i)),
        ],
        out_specs=pl.BlockSpec((B, 3, Rk), lambda i: (0, 0, i)),
        out_shape=jax.ShapeDtypeStruct((B, 3, HW), jnp.float32),
    )(resp, v_t)
    return out.reshape(B, 3, H, W)


# speedup vs baseline: 4.6032x; 4.6032x over previous
import functools

import jax
import jax.numpy as jnp
from jax.experimental import pallas as pl

EPS = 1e-8


def _interp_body(x_ref, y_ref, xi_ref, v_ref, out_ref):
    B = xi_ref.shape[0]
    n = x_ref.shape[1]
    resp = [[None] * 3 for _ in range(B)]
    for c in range(3):
        xp = x_ref[c, 0]
        y0 = y_ref[c, 0]
        yp = y0
        xis = [xi_ref[b, c] for b in range(B)]
        accs = [y0 for _ in range(B)]
        for k in range(1, n):
            xk = x_ref[c, k]
            yk = y_ref[c, k]
            dxe = (xk - xp) + EPS
            dyr = (yk - yp) / dxe
            for b in range(B):
                t = xis[b] - xp
                if k == 1:
                    u = jnp.minimum(t, dxe)
                elif k == n - 1:
                    u = jnp.maximum(t, 0.0)
                else:
                    u = jnp.clip(t, 0.0, dxe)
                accs[b] = accs[b] + u * dyr
            xp = xk
            yp = yk
        for b in range(B):
            resp[b][c] = accs[b]
    for b in range(B):
        for d in range(3):
            o = (resp[b][0] * v_ref[0, d]
                 + resp[b][1] * v_ref[1, d]
                 + resp[b][2] * v_ref[2, d])
            out_ref[b, d] = jnp.clip(o, 0.0, 1.0)


@functools.partial(jax.jit, static_argnames=())
def kernel(input_image, V, x_data, y_data):
    B = input_image.shape[0]
    _, H, W, n = x_data.shape
    HW = H * W
    R = min(2048, HW)
    grid = HW // R

    x_t = jnp.transpose(x_data.reshape(3, HW, n), (0, 2, 1))
    y_t = jnp.transpose(y_data.reshape(3, HW, n), (0, 2, 1))
    v_t = jnp.transpose(V.reshape(HW, 3, 3), (1, 2, 0))
    xi = input_image.reshape(B, 3, HW)

    out = pl.pallas_call(
        _interp_body,
        grid=(grid,),
        in_specs=[
            pl.BlockSpec((3, n, R), lambda i: (0, 0, i)),
            pl.BlockSpec((3, n, R), lambda i: (0, 0, i)),
            pl.BlockSpec((B, 3, R), lambda i: (0, 0, i)),
            pl.BlockSpec((3, 3, R), lambda i: (0, 0, i)),
        ],
        out_specs=pl.BlockSpec((B, 3, R), lambda i: (0, 0, i)),
        out_shape=jax.ShapeDtypeStruct((B, 3, HW), jnp.float32),
    )(x_t, y_t, xi, v_t)
    return out.reshape(B, 3, H, W)


# speedup vs baseline: 5.6361x; 1.2244x over previous
import functools

import jax
import jax.numpy as jnp
from jax.experimental import pallas as pl

EPS = 1e-8


def _interp_body(x_ref, y_ref, xi_ref, v_ref, out_ref):
    B = xi_ref.shape[0]
    n = x_ref.shape[1]
    resp = [[None] * 3 for _ in range(B)]
    for c in range(3):
        xp = x_ref[c, 0]
        y0 = y_ref[c, 0]
        yp = y0
        xis = [xi_ref[b, c] for b in range(B)]
        accs = [y0 for _ in range(B)]
        for k in range(1, n):
            xk = x_ref[c, k]
            yk = y_ref[c, k]
            dxe = (xk - xp) + EPS
            dyr = (yk - yp) / dxe
            for b in range(B):
                t = xis[b] - xp
                if k == 1:
                    u = jnp.minimum(t, dxe)
                elif k == n - 1:
                    u = jnp.maximum(t, 0.0)
                else:
                    u = jnp.clip(t, 0.0, dxe)
                accs[b] = accs[b] + u * dyr
            xp = xk
            yp = yk
        for b in range(B):
            resp[b][c] = accs[b]
    for b in range(B):
        for d in range(3):
            o = (resp[b][0] * v_ref[0, d]
                 + resp[b][1] * v_ref[1, d]
                 + resp[b][2] * v_ref[2, d])
            out_ref[b, d] = jnp.clip(o, 0.0, 1.0)


@functools.partial(jax.jit, static_argnames=())
def kernel(input_image, V, x_data, y_data):
    B = input_image.shape[0]
    _, H, W, n = x_data.shape
    HW = H * W
    R = min(8192, HW)
    grid = HW // R

    x_t = jnp.transpose(x_data.reshape(3, HW, n), (0, 2, 1))
    y_t = jnp.transpose(y_data.reshape(3, HW, n), (0, 2, 1))
    v_t = jnp.transpose(V.reshape(HW, 3, 3), (1, 2, 0))
    xi = input_image.reshape(B, 3, HW)

    out = pl.pallas_call(
        _interp_body,
        grid=(grid,),
        in_specs=[
            pl.BlockSpec((3, n, R), lambda i: (0, 0, i)),
            pl.BlockSpec((3, n, R), lambda i: (0, 0, i)),
            pl.BlockSpec((B, 3, R), lambda i: (0, 0, i)),
            pl.BlockSpec((3, 3, R), lambda i: (0, 0, i)),
        ],
        out_specs=pl.BlockSpec((B, 3, R), lambda i: (0, 0, i)),
        out_shape=jax.ShapeDtypeStruct((B, 3, HW), jnp.float32),
    )(x_t, y_t, xi, v_t)
    return out.reshape(B, 3, H, W)


# speedup vs baseline: 5.8463x; 1.0373x over previous
import functools

import jax
import jax.numpy as jnp
from jax.experimental import pallas as pl

EPS = 1e-8


def _interp_body(x_ref, y_ref, xi_ref, v_ref, out_ref):
    B = xi_ref.shape[0]
    n = x_ref.shape[1]
    resp = [[None] * 3 for _ in range(B)]
    for c in range(3):
        xp = x_ref[c, 0]
        y0 = y_ref[c, 0]
        yp = y0
        xis = [xi_ref[b, c] for b in range(B)]
        accs = [y0 for _ in range(B)]
        for k in range(1, n):
            xk = x_ref[c, k]
            yk = y_ref[c, k]
            dxe = (xk - xp) + EPS
            dyr = (yk - yp) / dxe
            for b in range(B):
                t = xis[b] - xp
                if k == 1:
                    u = jnp.minimum(t, dxe)
                elif k == n - 1:
                    u = jnp.maximum(t, 0.0)
                else:
                    u = jnp.clip(t, 0.0, dxe)
                accs[b] = accs[b] + u * dyr
            xp = xk
            yp = yk
        for b in range(B):
            resp[b][c] = accs[b]
    for b in range(B):
        for d in range(3):
            o = (resp[b][0] * v_ref[0, d]
                 + resp[b][1] * v_ref[1, d]
                 + resp[b][2] * v_ref[2, d])
            out_ref[b, d] = jnp.clip(o, 0.0, 1.0)


@functools.partial(jax.jit, static_argnames=())
def kernel(input_image, V, x_data, y_data):
    B = input_image.shape[0]
    _, H, W, n = x_data.shape
    HW = H * W
    R = min(16384, HW)
    grid = HW // R

    x_t = jnp.transpose(x_data.reshape(3, HW, n), (0, 2, 1))
    y_t = jnp.transpose(y_data.reshape(3, HW, n), (0, 2, 1))
    v_t = jnp.transpose(V.reshape(HW, 3, 3), (1, 2, 0))
    xi = input_image.reshape(B, 3, HW)

    out = pl.pallas_call(
        _interp_body,
        grid=(grid,),
        in_specs=[
            pl.BlockSpec((3, n, R), lambda i: (0, 0, i)),
            pl.BlockSpec((3, n, R), lambda i: (0, 0, i)),
            pl.BlockSpec((B, 3, R), lambda i: (0, 0, i)),
            pl.BlockSpec((3, 3, R), lambda i: (0, 0, i)),
        ],
        out_specs=pl.BlockSpec((B, 3, R), lambda i: (0, 0, i)),
        out_shape=jax.ShapeDtypeStruct((B, 3, HW), jnp.float32),
    )(x_t, y_t, xi, v_t)
    return out.reshape(B, 3, H, W)


# speedup vs baseline: 5.8801x; 1.0058x over previous
import functools

import jax
import jax.numpy as jnp
from jax.experimental import pallas as pl

EPS = 1e-8


def _interp_body(x_ref, y_ref, xi_ref, v_ref, out_ref):
    B = xi_ref.shape[0]
    n = x_ref.shape[1]
    resp = [[None] * 3 for _ in range(B)]
    for c in range(3):
        xp = x_ref[c, 0]
        y0 = y_ref[c, 0]
        yp = y0
        xis = [xi_ref[b, c] for b in range(B)]
        accs = [y0 for _ in range(B)]
        for k in range(1, n):
            xk = x_ref[c, k]
            yk = y_ref[c, k]
            dxe = (xk - xp) + EPS
            dyr = (yk - yp) / dxe
            for b in range(B):
                t = xis[b] - xp
                if k == 1:
                    u = jnp.minimum(t, dxe)
                elif k == n - 1:
                    u = jnp.maximum(t, 0.0)
                else:
                    u = jnp.clip(t, 0.0, dxe)
                accs[b] = accs[b] + u * dyr
            xp = xk
            yp = yk
        for b in range(B):
            resp[b][c] = accs[b]
    for b in range(B):
        for d in range(3):
            o = (resp[b][0] * v_ref[0, d]
                 + resp[b][1] * v_ref[1, d]
                 + resp[b][2] * v_ref[2, d])
            out_ref[b, d] = jnp.clip(o, 0.0, 1.0)


@functools.partial(jax.jit, static_argnames=())
def kernel(input_image, V, x_data, y_data):
    B = input_image.shape[0]
    _, H, W, n = x_data.shape
    HW = H * W
    R = min(32768, HW)
    grid = HW // R

    x_t = jnp.transpose(x_data.reshape(3, HW, n), (0, 2, 1))
    y_t = jnp.transpose(y_data.reshape(3, HW, n), (0, 2, 1))
    v_t = jnp.transpose(V.reshape(HW, 3, 3), (1, 2, 0))
    xi = input_image.reshape(B, 3, HW)

    out = pl.pallas_call(
        _interp_body,
        grid=(grid,),
        in_specs=[
            pl.BlockSpec((3, n, R), lambda i: (0, 0, i)),
            pl.BlockSpec((3, n, R), lambda i: (0, 0, i)),
            pl.BlockSpec((B, 3, R), lambda i: (0, 0, i)),
            pl.BlockSpec((3, 3, R), lambda i: (0, 0, i)),
        ],
        out_specs=pl.BlockSpec((B, 3, R), lambda i: (0, 0, i)),
        out_shape=jax.ShapeDtypeStruct((B, 3, HW), jnp.float32),
    )(x_t, y_t, xi, v_t)
    return out.reshape(B, 3, H, W)
